# baseline (speedup 1.0000x reference)
import jax
import jax.numpy as jnp
from jax.experimental import pallas as pl

_BATCH = 4096
_SEQ = 200
_D = 64
_ROW = _SEQ * _D
_BLOCK_B = 64


def _add_kernel(x_ref, emb_ref, out_ref):
    out_ref[...] = x_ref[...] + emb_ref[...]


def kernel(x, embedding):
    b, s, d = x.shape
    x2 = x.reshape(b, s * d)
    emb2 = embedding.reshape(1, s * d)
    out = pl.pallas_call(
        _add_kernel,
        grid=(b // _BLOCK_B,),
        in_specs=[
            pl.BlockSpec((_BLOCK_B, s * d), lambda i: (i, 0)),
            pl.BlockSpec((1, s * d), lambda i: (0, 0)),
        ],
        out_specs=pl.BlockSpec((_BLOCK_B, s * d), lambda i: (i, 0)),
        out_shape=jax.ShapeDtypeStruct((b, s * d), x.dtype),
    )(x2, emb2)
    return out.reshape(b, s, d)
